# baseline (speedup 1.0000x reference)
import functools

import jax
import jax.numpy as jnp
from jax import lax
from jax.experimental import pallas as pl
from jax.experimental.pallas import tpu as pltpu
from jax.experimental.pallas import tpu_sc as plsc

TOKENS = 32768
HIDDEN = 1024
NUM_EXPERTS = 8
TOP_K = 2
AUX_LOSS_COEF = 0.01

BLOCK_M = 2048

NC = 2
NS = 16
NW = NC * NS
TPW = TOKENS // NW
NV = TPW // 16
FM = 2 * NUM_EXPERTS * 16


def _matmul_kernel(xa_ref, xb_ref, w_ref, b_ref, out_ref):
    la = jax.lax.dot_general(
        w_ref[...], xa_ref[...], (((0,), (1,)), ((), ())),
        preferred_element_type=jnp.float32)
    lb = jax.lax.dot_general(
        w_ref[...], xb_ref[...], (((0,), (1,)), ((), ())),
        preferred_element_type=jnp.float32)
    out_ref[0] = la + b_ref[...]
    out_ref[1] = lb + b_ref[...]


def _gate_logits_t(x, W, b):
    grid = TOKENS // BLOCK_M
    return pl.pallas_call(
        _matmul_kernel,
        grid=(grid,),
        in_specs=[
            pl.BlockSpec((BLOCK_M // 2, HIDDEN), lambda i: (2 * i, 0)),
            pl.BlockSpec((BLOCK_M // 2, HIDDEN), lambda i: (2 * i + 1, 0)),
            pl.BlockSpec((HIDDEN, NUM_EXPERTS), lambda i: (0, 0)),
            pl.BlockSpec((NUM_EXPERTS, 1), lambda i: (0, 0)),
        ],
        out_specs=pl.BlockSpec((2, NUM_EXPERTS, TPW), lambda i: (i, 0, 0)),
        out_shape=jax.ShapeDtypeStruct((NW, NUM_EXPERTS, TPW), jnp.float32),
    )(x, x, W, b.reshape(NUM_EXPERTS, 1))


def _sc_router(logits_hbm, idx_hbm, scores_hbm, mask_hbm, fm_hbm,
               lg_v, idx_v, sc_v, mk_v, fm_v):
    wid = lax.axis_index("s") * NC + lax.axis_index("c")
    base = wid * TPW
    pltpu.sync_copy(logits_hbm.at[wid], lg_v)

    zeros_f = jnp.zeros((16,), jnp.float32)

    def body(j, accs):
        f_acc = list(accs[0])
        m_acc = list(accs[1])
        t0 = j * 16
        l = [lg_v[e, pl.ds(t0, 16)] for e in range(NUM_EXPERTS)]

        top1 = l[0]
        for e in range(1, NUM_EXPERTS):
            top1 = jnp.maximum(top1, l[e])
        idx1 = jnp.full((16,), NUM_EXPERTS, jnp.int32)
        for e in range(NUM_EXPERTS - 1, -1, -1):
            idx1 = jnp.where(l[e] == top1, jnp.int32(e), idx1)

        neg = jnp.float32(-jnp.inf)
        top2 = jnp.full((16,), neg, jnp.float32)
        for e in range(NUM_EXPERTS):
            le = jnp.where(idx1 == e, neg, l[e])
            top2 = jnp.maximum(top2, le)
        idx2 = jnp.full((16,), NUM_EXPERTS, jnp.int32)
        for e in range(NUM_EXPERTS - 1, -1, -1):
            hit = jnp.logical_and(l[e] == top2, idx1 != e)
            idx2 = jnp.where(hit, jnp.int32(e), idx2)

        e2 = jnp.exp(top2 - top1)
        s1 = 1.0 / (1.0 + e2)

        den = zeros_f
        p = []
        for e in range(NUM_EXPERTS):
            pe = jnp.exp(l[e] - top1)
            p.append(pe)
            den = den + pe
        inv = 1.0 / den

        for e in range(NUM_EXPERTS):
            m_acc[e] = m_acc[e] + p[e] * inv
            hot1 = jnp.where(idx1 == e, 1.0, 0.0)
            hot2 = jnp.where(idx2 == e, 1.0, 0.0)
            f_acc[e] = f_acc[e] + hot1 + hot2
            mk_v[pl.ds(e * TPW + t0, 16)] = hot1
            mk_v[pl.ds((NUM_EXPERTS + e) * TPW + t0, 16)] = hot2

        idx_v[pl.ds(t0, 16)] = idx1
        idx_v[pl.ds(TPW + t0, 16)] = idx2
        sc_v[pl.ds(t0, 16)] = s1
        sc_v[pl.ds(TPW + t0, 16)] = 1.0 - s1
        return (tuple(f_acc), tuple(m_acc))

    init = (tuple(jnp.zeros((16,), jnp.float32) for _ in range(NUM_EXPERTS)),
            tuple(jnp.zeros((16,), jnp.float32) for _ in range(NUM_EXPERTS)))
    f_acc, m_acc = lax.fori_loop(0, NV, body, init)
    for e in range(NUM_EXPERTS):
        fm_v[pl.ds(e * 16, 16)] = f_acc[e]
        fm_v[pl.ds((NUM_EXPERTS + e) * 16, 16)] = m_acc[e]

    for s in range(TOP_K):
        pltpu.sync_copy(idx_v.at[pl.ds(s * TPW, TPW)],
                        idx_hbm.at[pl.ds(s * TOKENS + base, TPW)])
        pltpu.sync_copy(sc_v.at[pl.ds(s * TPW, TPW)],
                        scores_hbm.at[pl.ds(s * TOKENS + base, TPW)])
    for r in range(TOP_K * NUM_EXPERTS):
        pltpu.sync_copy(mk_v.at[pl.ds(r * TPW, TPW)],
                        mask_hbm.at[pl.ds(r * TOKENS + base, TPW)])
    pltpu.sync_copy(fm_v, fm_hbm.at[pl.ds(wid * FM, FM)])


_sc_call = functools.partial(
    pl.kernel,
    mesh=plsc.VectorSubcoreMesh(core_axis_name="c", subcore_axis_name="s"),
    out_type=[
        jax.ShapeDtypeStruct((TOP_K * TOKENS,), jnp.int32),
        jax.ShapeDtypeStruct((TOP_K * TOKENS,), jnp.float32),
        jax.ShapeDtypeStruct((TOP_K * NUM_EXPERTS * TOKENS,), jnp.float32),
        jax.ShapeDtypeStruct((NW * FM,), jnp.float32),
    ],
    scratch_types=[
        pltpu.VMEM((NUM_EXPERTS, TPW), jnp.float32),
        pltpu.VMEM((TOP_K * TPW,), jnp.int32),
        pltpu.VMEM((TOP_K * TPW,), jnp.float32),
        pltpu.VMEM((TOP_K * NUM_EXPERTS * TPW,), jnp.float32),
        pltpu.VMEM((FM,), jnp.float32),
    ],
)(_sc_router)


@functools.partial(jax.jit, static_argnames=())
def kernel(x, W, b):
    logits_t = _gate_logits_t(x, W, b)
    idx_f, scores_f, mask_f, fm = _sc_call(logits_t)
    expert_indices = idx_f.reshape(TOP_K, TOKENS).T
    scores = scores_f.reshape(TOP_K, TOKENS).T
    expert_mask = mask_f.reshape(TOP_K, NUM_EXPERTS, TOKENS).transpose(2, 0, 1)
    fm2 = fm.reshape(NW, 2 * NUM_EXPERTS, 16)
    f_i = jnp.sum(fm2[:, :NUM_EXPERTS, :], axis=(0, 2)) / (TOKENS * TOP_K)
    m_i = jnp.sum(fm2[:, NUM_EXPERTS:, :], axis=(0, 2)) / TOKENS
    aux = (AUX_LOSS_COEF / NUM_EXPERTS) * jnp.sum(f_i * m_i)
    return expert_indices, scores, expert_mask, aux


# speedup vs baseline: 1.6500x; 1.6500x over previous
import functools

import jax
import jax.numpy as jnp
from jax.experimental import pallas as pl
from jax.experimental.pallas import tpu as pltpu

TOKENS = 32768
HIDDEN = 1024
NUM_EXPERTS = 8
TOP_K = 2
AUX_LOSS_COEF = 0.01

BLOCK_M = 2048


def _router_kernel(xa_ref, xb_ref, w_ref, b_ref, idx_ref, scores_ref,
                   mask_ref, aux_ref, fsum_ref, msum_ref):
    step = pl.program_id(0)
    nsteps = pl.num_programs(0)

    la = jax.lax.dot_general(
        w_ref[...], xa_ref[...], (((0,), (1,)), ((), ())),
        preferred_element_type=jnp.float32)
    lb = jax.lax.dot_general(
        w_ref[...], xb_ref[...], (((0,), (1,)), ((), ())),
        preferred_element_type=jnp.float32)
    logits = jnp.concatenate([la, lb], axis=1) + b_ref[...]

    eids = jax.lax.broadcasted_iota(jnp.int32, logits.shape, 0)
    top1 = jnp.max(logits, axis=0, keepdims=True)
    idx1 = jnp.min(jnp.where(logits == top1, eids, NUM_EXPERTS),
                   axis=0, keepdims=True)
    hit1 = eids == idx1
    mask1 = hit1.astype(jnp.float32)
    masked = jnp.where(hit1, -jnp.inf, logits)
    top2 = jnp.max(masked, axis=0, keepdims=True)
    idx2 = jnp.min(jnp.where(masked == top2, eids, NUM_EXPERTS),
                   axis=0, keepdims=True)
    mask2 = (eids == idx2).astype(jnp.float32)

    idx_ref[...] = jnp.concatenate([idx1, idx2], axis=0)

    e2 = jnp.exp(top2 - top1)
    s1 = 1.0 / (1.0 + e2)
    scores_ref[...] = jnp.concatenate([s1, 1.0 - s1], axis=0)

    mask_ref[...] = jnp.concatenate([mask1, mask2], axis=0)

    p = jnp.exp(logits - top1)
    p = p / jnp.sum(p, axis=0, keepdims=True)

    f_part = jnp.sum(mask1 + mask2, axis=1, keepdims=True)
    m_part = jnp.sum(p, axis=1, keepdims=True)

    @pl.when(step == 0)
    def _init():
        fsum_ref[...] = jnp.zeros_like(fsum_ref)
        msum_ref[...] = jnp.zeros_like(msum_ref)

    fsum_ref[...] += f_part
    msum_ref[...] += m_part

    @pl.when(step == nsteps - 1)
    def _fin():
        f_i = fsum_ref[...] / (TOKENS * TOP_K)
        m_i = msum_ref[...] / TOKENS
        aux_ref[...] = (AUX_LOSS_COEF / NUM_EXPERTS) * jnp.sum(
            f_i * m_i, keepdims=True).reshape(1, 1)


def _router_call(x, W, b):
    grid = TOKENS // BLOCK_M
    return pl.pallas_call(
        _router_kernel,
        grid=(grid,),
        in_specs=[
            pl.BlockSpec((BLOCK_M // 2, HIDDEN), lambda i: (2 * i, 0)),
            pl.BlockSpec((BLOCK_M // 2, HIDDEN), lambda i: (2 * i + 1, 0)),
            pl.BlockSpec((HIDDEN, NUM_EXPERTS), lambda i: (0, 0)),
            pl.BlockSpec((NUM_EXPERTS, 1), lambda i: (0, 0)),
        ],
        out_specs=[
            pl.BlockSpec((TOP_K, BLOCK_M), lambda i: (0, i)),
            pl.BlockSpec((TOP_K, BLOCK_M), lambda i: (0, i)),
            pl.BlockSpec((TOP_K * NUM_EXPERTS, BLOCK_M), lambda i: (0, i)),
            pl.BlockSpec((1, 1), lambda i: (0, 0)),
        ],
        out_shape=[
            jax.ShapeDtypeStruct((TOP_K, TOKENS), jnp.int32),
            jax.ShapeDtypeStruct((TOP_K, TOKENS), jnp.float32),
            jax.ShapeDtypeStruct((TOP_K * NUM_EXPERTS, TOKENS), jnp.float32),
            jax.ShapeDtypeStruct((1, 1), jnp.float32),
        ],
        scratch_shapes=[
            pltpu.VMEM((NUM_EXPERTS, 1), jnp.float32),
            pltpu.VMEM((NUM_EXPERTS, 1), jnp.float32),
        ],
    )(x, x, W, b.reshape(NUM_EXPERTS, 1))


@functools.partial(jax.jit, static_argnames=())
def kernel(x, W, b):
    idx_t, scores_t, mask_t, aux = _router_call(x, W, b)
    expert_indices = idx_t.T
    scores = scores_t.T
    expert_mask = mask_t.reshape(TOP_K, NUM_EXPERTS, TOKENS).transpose(2, 0, 1)
    return expert_indices, scores, expert_mask, aux[0, 0]
